# baseline (speedup 1.0000x reference)
import jax
import jax.numpy as jnp
from jax.experimental import pallas as pl


def _add_block(x_ref, t_ref, o_ref):
    o_ref[...] = x_ref[...] + t_ref[...]


def kernel(x, pos_table):
    B, S, D = x.shape
    BS = 2048
    NSB = S // BS
    x2 = x.reshape(B * S, D)
    out = pl.pallas_call(
        _add_block,
        grid=(NSB, B),
        in_specs=[
            pl.BlockSpec((BS, D), lambda i, b: (b * NSB + i, 0)),
            pl.BlockSpec((BS, D), lambda i, b: (i, 0)),
        ],
        out_specs=pl.BlockSpec((BS, D), lambda i, b: (b * NSB + i, 0)),
        out_shape=jax.ShapeDtypeStruct((B * S, D), x.dtype),
    )(x2, pos_table[:S])
    return out.reshape(B, S, D)
